# baseline (speedup 1.0000x reference)
import functools

import jax
import jax.numpy as jnp
from jax import lax
from jax.experimental import pallas as pl
from jax.experimental.pallas import tpu as pltpu
from jax.experimental.pallas import tpu_sc as plsc

NUM_CORES = 2
NUM_SUBCORES = 16
NW = NUM_CORES * NUM_SUBCORES
LANES = 16
WIN = 128
PACK = 4


def _sc_gmf(uidx_hbm, iidx_hbm, utab_hbm, itab_hbm, w_hbm, b_hbm, out_hbm,
            uidx_v, iidx_v, uphys_v, iphys_v, u_buf, i_buf, w_v, b_v, out_v,
            sem, *, bpw, d):
  n_win = bpw // WIN
  wid = lax.axis_index("s") * NUM_CORES + lax.axis_index("c")
  base = wid * bpw

  pltpu.sync_copy(uidx_hbm.at[pl.ds(base, bpw)], uidx_v)
  pltpu.sync_copy(iidx_hbm.at[pl.ds(base, bpw)], iidx_v)
  pltpu.sync_copy(w_hbm, w_v)
  pltpu.sync_copy(b_hbm, b_v)

  for t in range(bpw // LANES):
    s = pl.ds(t * LANES, LANES)
    uphys_v[s] = lax.shift_right_logical(uidx_v[s], 2)
    iphys_v[s] = lax.shift_right_logical(iidx_v[s], 2)

  lanes = lax.iota(jnp.int32, LANES)
  b_vec = b_v[...]
  w_chunks = [w_v[pl.ds(k * LANES, LANES)] for k in range(d // LANES)]
  wb = [jnp.broadcast_to(w_chunks[dd // LANES][dd % LANES], (LANES,))
        for dd in range(d)]

  for win in range(n_win):
    rows = pl.ds(win * WIN, WIN)
    cu = pltpu.async_copy(utab_hbm.at[uphys_v.at[rows]], u_buf, sem)
    ci = pltpu.async_copy(itab_hbm.at[iphys_v.at[rows]], i_buf, sem)
    cu.wait()
    ci.wait()

    def body(g, carry):
      s = pl.ds(win * WIN + g * LANES, LANES)
      row = g * LANES + lanes
      uoff = lax.shift_left(jnp.bitwise_and(uidx_v[s], 3), 5)
      ioff = lax.shift_left(jnp.bitwise_and(iidx_v[s], 3), 5)
      acc = jnp.zeros((LANES,), jnp.float32)
      for dd in range(d):
        uv = plsc.load_gather(u_buf, [row, uoff + dd])
        iv = plsc.load_gather(i_buf, [row, ioff + dd])
        acc = acc + (uv * iv) * wb[dd]
      logit = acc + b_vec
      sig = 1.0 / (1.0 + jnp.exp(-logit))
      out_v[pl.ds(win * WIN + g * LANES, LANES)] = sig
      return carry

    lax.fori_loop(0, WIN // LANES, body, 0)

  pltpu.sync_copy(out_v, out_hbm.at[pl.ds(base, bpw)])


def kernel(user_indices, item_indices, user_table, item_table, W, b):
  B = user_indices.shape[0]
  V, D = user_table.shape
  bpw = B // NW

  uidx = user_indices.astype(jnp.int32)
  iidx = item_indices.astype(jnp.int32)
  ut = user_table.reshape(V * D // 128, 128)
  it = item_table.reshape(V * D // 128, 128)
  w_flat = W.reshape(D).astype(jnp.float32)
  b_vec = jnp.broadcast_to(b.astype(jnp.float32), (LANES,))

  mesh = plsc.VectorSubcoreMesh(core_axis_name="c", subcore_axis_name="s")
  sc = functools.partial(
      pl.kernel,
      mesh=mesh,
      compiler_params=pltpu.CompilerParams(
          needs_layout_passes=False, use_tc_tiling_on_sc=True),
      out_type=jax.ShapeDtypeStruct((B,), jnp.float32),
      scratch_types=[
          pltpu.VMEM((bpw,), jnp.int32),
          pltpu.VMEM((bpw,), jnp.int32),
          pltpu.VMEM((bpw,), jnp.int32),
          pltpu.VMEM((bpw,), jnp.int32),
          pltpu.VMEM((WIN, 128), jnp.float32),
          pltpu.VMEM((WIN, 128), jnp.float32),
          pltpu.VMEM((D,), jnp.float32),
          pltpu.VMEM((LANES,), jnp.float32),
          pltpu.VMEM((bpw,), jnp.float32),
          pltpu.SemaphoreType.DMA,
      ],
  )(functools.partial(_sc_gmf, bpw=bpw, d=D))

  out = sc(uidx, iidx, ut, it, w_flat, b_vec)
  return out.reshape(B, 1)
